# baseline (speedup 1.0000x reference)
import jax
import jax.numpy as jnp
from jax.experimental import pallas as pl
from jax.experimental.pallas import tpu as pltpu

N = 1024
OBS = 128
HID = 256
H = 256
C = N // 2


def _encoder_body(obs_ref, hid_ref, mat_ref,
                  obs_cW_ref, obs_cb_ref, obs_f1W_ref, obs_f1b_ref,
                  obs_f2W_ref, obs_f2b_ref,
                  hid_cW_ref, hid_cb_ref, hid_f1W_ref, hid_f1b_ref,
                  hid_f2W_ref, hid_f2b_ref,
                  gru_Wih_ref, gru_Whh_ref, gru_bih_ref, gru_bhh_ref,
                  enc_W_ref, enc_b_ref,
                  latent_ref, next_hid_ref,
                  dinv_ref, mfA_ref):
    k = pl.program_id(0)
    mf = mat_ref[...].astype(jnp.bfloat16)

    def dot16(a, b):
        return jnp.dot(a.astype(jnp.bfloat16), b.astype(jnp.bfloat16),
                       preferred_element_type=jnp.float32)

    def dot16_t(a, b):
        return jax.lax.dot_general(
            a.astype(jnp.bfloat16), b.astype(jnp.bfloat16),
            (((0,), (0,)), ((), ())), preferred_element_type=jnp.float32)

    ones = jnp.ones((N, 1), jnp.bfloat16)
    deg = dot16_t(mf, ones) + 1.0
    dinv_ref[pl.ds(k * C, C), :] = jax.lax.rsqrt(deg)

    @pl.when(k == 0)
    def _park():
        mfA_ref[...] = mf

    @pl.when(k == 1)
    def _tail():
        dinv = dinv_ref[...]
        s_o = dinv * dot16(obs_ref[...], obs_cW_ref[...])
        s_h = dinv * dot16(hid_ref[...], hid_cW_ref[...])

        def dot16_rt(a, b):
            return jax.lax.dot_general(
                a.astype(jnp.bfloat16), b.astype(jnp.bfloat16),
                (((1,), (1,)), ((), ())), preferred_element_type=jnp.float32)

        for half, mfh in ((0, mfA_ref[...]), (1, mf)):
            sl = slice(half * C, (half + 1) * C)
            dv = dinv[sl]

            def branch(s, cb, f1W, f1b, f2W, f2b):
                agg = dot16_t(mfh, s) + s[sl]
                h = jnp.maximum(dv * agg + cb, 0.0)
                h = jnp.maximum(dot16(h, f1W) + f1b, 0.0)
                return dot16(h, f2W) + f2b

            phi = branch(s_o, obs_cb_ref[...],
                         obs_f1W_ref[...], obs_f1b_ref[...],
                         obs_f2W_ref[...], obs_f2b_ref[...])
            psi = branch(s_h, hid_cb_ref[...],
                         hid_f1W_ref[...], hid_f1b_ref[...],
                         hid_f2W_ref[...], hid_f2b_ref[...])

            gi = dot16_rt(phi, gru_Wih_ref[...]) + gru_bih_ref[...]
            gh = dot16_rt(psi, gru_Whh_ref[...]) + gru_bhh_ref[...]
            r = jax.nn.sigmoid(gi[:, :HID] + gh[:, :HID])
            z = jax.nn.sigmoid(gi[:, HID:2 * HID] + gh[:, HID:2 * HID])
            n = jnp.tanh(gi[:, 2 * HID:] + r * gh[:, 2 * HID:])
            next_hid = (1.0 - z) * n + z * psi

            latent_ref[sl, :] = dot16(next_hid, enc_W_ref[...]) + enc_b_ref[...]
            next_hid_ref[sl, :] = next_hid


def kernel(obs, hidden_states, matrix,
           obs_cW, obs_cb, obs_f1W, obs_f1b, obs_f2W, obs_f2b,
           hid_cW, hid_cb, hid_f1W, hid_f1b, hid_f2W, hid_f2b,
           gru_Wih, gru_Whh, gru_bih, gru_bhh,
           enc_W, enc_b):
    colchunk = lambda k: (0, k)
    whole = lambda k: (0, 0)
    vec = lambda k: (0,)
    latent, next_hid = pl.pallas_call(
        _encoder_body,
        grid=(2,),
        in_specs=[
            pl.BlockSpec((N, OBS), whole),
            pl.BlockSpec((N, HID), whole),
            pl.BlockSpec((N, C), colchunk),
            pl.BlockSpec((OBS, H), whole),
            pl.BlockSpec((H,), vec),
            pl.BlockSpec((H, H), whole),
            pl.BlockSpec((H,), vec),
            pl.BlockSpec((H, OBS), whole),
            pl.BlockSpec((OBS,), vec),
            pl.BlockSpec((HID, H), whole),
            pl.BlockSpec((H,), vec),
            pl.BlockSpec((H, H), whole),
            pl.BlockSpec((H,), vec),
            pl.BlockSpec((H, HID), whole),
            pl.BlockSpec((HID,), vec),
            pl.BlockSpec((3 * HID, OBS), whole),
            pl.BlockSpec((3 * HID, HID), whole),
            pl.BlockSpec((3 * HID,), vec),
            pl.BlockSpec((3 * HID,), vec),
            pl.BlockSpec((HID, H), whole),
            pl.BlockSpec((H,), vec),
        ],
        out_specs=(
            pl.BlockSpec((N, H), whole),
            pl.BlockSpec((N, HID), whole),
        ),
        out_shape=(
            jax.ShapeDtypeStruct((N, H), jnp.float32),
            jax.ShapeDtypeStruct((N, HID), jnp.float32),
        ),
        scratch_shapes=[
            pltpu.VMEM((N, 1), jnp.float32),
            pltpu.VMEM((N, C), jnp.bfloat16),
        ],
    )(obs, hidden_states, matrix,
      obs_cW, obs_cb, obs_f1W, obs_f1b, obs_f2W, obs_f2b,
      hid_cW, hid_cb, hid_f1W, hid_f1b, hid_f2W, hid_f2b,
      gru_Wih, gru_Whh, gru_bih, gru_bhh,
      enc_W, enc_b)
    return (latent, next_hid)
